# baseline (speedup 1.0000x reference)
import functools

import jax
import jax.numpy as jnp
from jax import lax
from jax.experimental import pallas as pl
from jax.experimental.pallas import tpu as pltpu
from jax.experimental.pallas import tpu_sc as plsc

VOCAB_ROWS = 100000
DIM = 64

_NUM_CORES = 2
_NUM_SUBCORES = 16
_NUM_WORKERS = _NUM_CORES * _NUM_SUBCORES
_CHUNK = 3128
_LAST = VOCAB_ROWS - (_NUM_WORKERS - 1) * _CHUNK

_mesh = plsc.VectorSubcoreMesh(core_axis_name="c", subcore_axis_name="s")


@functools.partial(
    pl.kernel,
    mesh=_mesh,
    out_type=jax.ShapeDtypeStruct((VOCAB_ROWS, DIM), jnp.float32),
)
def _copy_kernel(in_hbm, out_hbm):
    wid = lax.axis_index("s") * _NUM_CORES + lax.axis_index("c")
    base = pl.multiple_of(wid * _CHUNK, 8)

    @pl.when(wid < _NUM_WORKERS - 1)
    def _():
        pltpu.sync_copy(
            in_hbm.at[pl.ds(base, _CHUNK)],
            out_hbm.at[pl.ds(base, _CHUNK)],
        )

    @pl.when(wid == _NUM_WORKERS - 1)
    def _():
        last_base = (_NUM_WORKERS - 1) * _CHUNK
        pltpu.sync_copy(
            in_hbm.at[pl.ds(last_base, _LAST)],
            out_hbm.at[pl.ds(last_base, _LAST)],
        )


def kernel(embeddings):
    return _copy_kernel(embeddings)


# speedup vs baseline: 9.8903x; 9.8903x over previous
import functools

import jax
import jax.numpy as jnp
from jax import lax
from jax.experimental import pallas as pl
from jax.experimental.pallas import tpu as pltpu
from jax.experimental.pallas import tpu_sc as plsc

VOCAB_ROWS = 100000
DIM = 64
ROWS = VOCAB_ROWS * DIM // 128
LANES = 128

_NUM_CORES = 2
_NUM_SUBCORES = 16
_NUM_WORKERS = _NUM_CORES * _NUM_SUBCORES
_WCHUNK = 1568
_LAST = ROWS - (_NUM_WORKERS - 1) * _WCHUNK
_CROWS = 392


def _chunk_sizes(total):
    sizes = []
    while total > 0:
        sizes.append(min(_CROWS, total))
        total -= sizes[-1]
    return sizes


@functools.partial(
    pl.kernel,
    mesh=plsc.VectorSubcoreMesh(core_axis_name="c", subcore_axis_name="s"),
    out_type=jax.ShapeDtypeStruct((ROWS, LANES), jnp.float32),
    scratch_types=[
        pltpu.VMEM((_CROWS, LANES), jnp.float32),
        pltpu.VMEM((_CROWS, LANES), jnp.float32),
        pltpu.SemaphoreType.DMA,
        pltpu.SemaphoreType.DMA,
        pltpu.SemaphoreType.DMA,
        pltpu.SemaphoreType.DMA,
    ],
)
def _copy_kernel(in_hbm, out_hbm, buf0, buf1, si0, si1, so0, so1):
    wid = lax.axis_index("s") * _NUM_CORES + lax.axis_index("c")
    base = pl.multiple_of(wid * _WCHUNK, 8)
    bufs = (buf0, buf1)
    sin = (si0, si1)
    sout = (so0, so1)

    def copy_range(start, total):
        sizes = _chunk_sizes(total)
        n = len(sizes)
        h_in = [None, None]
        h_out = [None, None]
        offs = []
        off = 0
        for sz in sizes:
            offs.append(off)
            off += sz
        h_in[0] = pltpu.async_copy(
            in_hbm.at[pl.ds(start + offs[0], sizes[0])],
            bufs[0].at[pl.ds(0, sizes[0])], sin[0])
        for i in range(n):
            b = i % 2
            h_in[b].wait()
            if i + 1 < n:
                if h_out[1 - b] is not None:
                    h_out[1 - b].wait()
                h_in[1 - b] = pltpu.async_copy(
                    in_hbm.at[pl.ds(start + offs[i + 1], sizes[i + 1])],
                    bufs[1 - b].at[pl.ds(0, sizes[i + 1])], sin[1 - b])
            h_out[b] = pltpu.async_copy(
                bufs[b].at[pl.ds(0, sizes[i])],
                out_hbm.at[pl.ds(start + offs[i], sizes[i])], sout[b])
        for h in h_out:
            if h is not None:
                h.wait()

    @pl.when(wid < _NUM_WORKERS - 1)
    def _():
        copy_range(base, _WCHUNK)

    @pl.when(wid == _NUM_WORKERS - 1)
    def _():
        copy_range((_NUM_WORKERS - 1) * _WCHUNK, _LAST)


def kernel(embeddings):
    flat = embeddings.reshape(ROWS, LANES)
    return _copy_kernel(flat).reshape(VOCAB_ROWS, DIM)


# speedup vs baseline: 12.9714x; 1.3115x over previous
import functools

import jax
import jax.numpy as jnp
from jax import lax
from jax.experimental import pallas as pl
from jax.experimental.pallas import tpu as pltpu
from jax.experimental.pallas import tpu_sc as plsc

VOCAB_ROWS = 100000
DIM = 64

_NUM_CORES = 2
_NUM_SUBCORES = 16
_NUM_WORKERS = _NUM_CORES * _NUM_SUBCORES
_WCHUNK = 3128
_LAST = VOCAB_ROWS - (_NUM_WORKERS - 1) * _WCHUNK
_CROWS = 504


def _chunk_sizes(total):
    sizes = []
    while total > 0:
        sizes.append(min(_CROWS, total))
        total -= sizes[-1]
    return sizes


@functools.partial(
    pl.kernel,
    mesh=plsc.VectorSubcoreMesh(core_axis_name="c", subcore_axis_name="s"),
    out_type=jax.ShapeDtypeStruct((VOCAB_ROWS, DIM), jnp.float32),
    scratch_types=[
        pltpu.VMEM((_CROWS, DIM), jnp.float32),
        pltpu.VMEM((_CROWS, DIM), jnp.float32),
        pltpu.SemaphoreType.DMA,
        pltpu.SemaphoreType.DMA,
        pltpu.SemaphoreType.DMA,
        pltpu.SemaphoreType.DMA,
    ],
)
def _copy_kernel(in_hbm, out_hbm, buf0, buf1, si0, si1, so0, so1):
    wid = lax.axis_index("s") * _NUM_CORES + lax.axis_index("c")
    base = pl.multiple_of(wid * _WCHUNK, 8)
    bufs = (buf0, buf1)
    sin = (si0, si1)
    sout = (so0, so1)

    def copy_range(start, total):
        sizes = _chunk_sizes(total)
        n = len(sizes)
        h_in = [None, None]
        h_out = [None, None]
        offs = []
        off = 0
        for sz in sizes:
            offs.append(off)
            off += sz
        h_in[0] = pltpu.async_copy(
            in_hbm.at[pl.ds(start + offs[0], sizes[0])],
            bufs[0].at[pl.ds(0, sizes[0])], sin[0])
        for i in range(n):
            b = i % 2
            h_in[b].wait()
            if i + 1 < n:
                if h_out[1 - b] is not None:
                    h_out[1 - b].wait()
                h_in[1 - b] = pltpu.async_copy(
                    in_hbm.at[pl.ds(start + offs[i + 1], sizes[i + 1])],
                    bufs[1 - b].at[pl.ds(0, sizes[i + 1])], sin[1 - b])
            h_out[b] = pltpu.async_copy(
                bufs[b].at[pl.ds(0, sizes[i])],
                out_hbm.at[pl.ds(start + offs[i], sizes[i])], sout[b])
        for h in h_out:
            if h is not None:
                h.wait()

    @pl.when(wid < _NUM_WORKERS - 1)
    def _():
        copy_range(base, _WCHUNK)

    @pl.when(wid == _NUM_WORKERS - 1)
    def _():
        copy_range((_NUM_WORKERS - 1) * _WCHUNK, _LAST)


def kernel(embeddings):
    return _copy_kernel(embeddings)
